# baseline (speedup 1.0000x reference)
import functools
import math

import jax
import jax.numpy as jnp
from jax import lax
from jax.experimental import pallas as pl
from jax.experimental.pallas import tpu as pltpu
from jax.experimental.pallas import tpu_sc as plsc

B = 16384
D = 64
H = 1024
BN_EPS = 1e-5
_BN_INV = float(1.0 / math.sqrt(1.0 + BN_EPS))

_NC, _NS = 2, 16
_NW = _NC * _NS
_BPW = B // _NW
_CH = 128
_NCH = _BPW // _CH


def _gather_body(u_hbm, m_hbm, u2_hbm, m2_hbm, ue_out, me_out,
                 uidx, midx, ub0, ub1, mb0, mb1, su0, su1, sm0, sm1):
    wid = lax.axis_index("s") * _NC + lax.axis_index("c")
    base = wid * _BPW
    pltpu.sync_copy(u_hbm.at[pl.ds(base, _BPW)], uidx)
    pltpu.sync_copy(m_hbm.at[pl.ds(base, _BPW)], midx)

    def _shift(i, carry):
        s = pl.ds(i * 16, 16)
        uidx[s] = uidx[s] >> 1
        midx[s] = midx[s] >> 1
        return carry

    lax.fori_loop(0, _BPW // 16, _shift, 0)

    ubufs, mbufs = (ub0, ub1), (mb0, mb1)
    usems, msems = (su0, su1), (sm0, sm1)

    def _start(k):
        b = k % 2
        cu = pltpu.async_copy(u2_hbm.at[uidx.at[pl.ds(k * _CH, _CH)]],
                              ubufs[b], usems[b])
        cm = pltpu.async_copy(m2_hbm.at[midx.at[pl.ds(k * _CH, _CH)]],
                              mbufs[b], msems[b])
        return cu, cm

    inflight = [_start(0), _start(1)]
    for k in range(_NCH):
        b = k % 2
        cu, cm = inflight[k]
        dst = pl.ds(base + k * _CH, _CH)
        cu.wait()
        pltpu.sync_copy(ubufs[b], ue_out.at[dst])
        cm.wait()
        pltpu.sync_copy(mbufs[b], me_out.at[dst])
        if k + 2 < _NCH:
            inflight.append(_start(k + 2))


@functools.cache
def _build_gather():
    mesh = plsc.VectorSubcoreMesh(core_axis_name="c", subcore_axis_name="s",
                                  num_cores=_NC, num_subcores=_NS)
    return pl.kernel(
        _gather_body,
        mesh=mesh,
        out_type=[jax.ShapeDtypeStruct((B, 2 * D), jnp.float32),
                  jax.ShapeDtypeStruct((B, 2 * D), jnp.float32)],
        scratch_types=[
            pltpu.VMEM((_BPW,), jnp.int32),
            pltpu.VMEM((_BPW,), jnp.int32),
            pltpu.VMEM((_CH, 2 * D), jnp.float32),
            pltpu.VMEM((_CH, 2 * D), jnp.float32),
            pltpu.VMEM((_CH, 2 * D), jnp.float32),
            pltpu.VMEM((_CH, 2 * D), jnp.float32),
            pltpu.SemaphoreType.DMA,
            pltpu.SemaphoreType.DMA,
            pltpu.SemaphoreType.DMA,
            pltpu.SemaphoreType.DMA,
        ],
    )


_BLK = 1024


def _mlp_body(ue2_ref, me2_ref, u_ref, m_ref, w1_ref, b1_ref, gamma_ref,
              beta_ref, w2_ref, b2_ref, out_ref):
    ue2, me2 = ue2_ref[...], me2_ref[...]
    odd_u = (u_ref[...] & 1) == 1
    odd_m = (m_ref[...] & 1) == 1
    xu = jnp.where(odd_u, ue2[:, D:], ue2[:, :D])
    xm = jnp.where(odd_m, me2[:, D:], me2[:, :D])
    x = jnp.concatenate([xu, xm], axis=1)
    h = jnp.dot(x, w1_ref[...], preferred_element_type=jnp.float32)
    h = jnp.maximum(h + b1_ref[...], 0.0)
    h = h * (gamma_ref[...] * _BN_INV) + beta_ref[...]
    out = jnp.sum(h * w2_ref[...], axis=1, keepdims=True) + b2_ref[...]
    out_ref[...] = out


_mlp = pl.pallas_call(
    _mlp_body,
    grid=(B // _BLK,),
    in_specs=[
        pl.BlockSpec((_BLK, 2 * D), lambda i: (i, 0)),
        pl.BlockSpec((_BLK, 2 * D), lambda i: (i, 0)),
        pl.BlockSpec((_BLK, 1), lambda i: (i, 0)),
        pl.BlockSpec((_BLK, 1), lambda i: (i, 0)),
        pl.BlockSpec((2 * D, H), lambda i: (0, 0)),
        pl.BlockSpec((1, H), lambda i: (0, 0)),
        pl.BlockSpec((1, H), lambda i: (0, 0)),
        pl.BlockSpec((1, H), lambda i: (0, 0)),
        pl.BlockSpec((1, H), lambda i: (0, 0)),
        pl.BlockSpec((1, 1), lambda i: (0, 0)),
    ],
    out_specs=pl.BlockSpec((_BLK, 1), lambda i: (i, 0)),
    out_shape=jax.ShapeDtypeStruct((B, 1), jnp.float32),
)


def kernel(u, m, u_emb, m_emb, W1, b1, gamma, beta, W2, b2):
    u2 = u_emb.reshape(u_emb.shape[0] // 2, 2 * D)
    m2 = m_emb.reshape(m_emb.shape[0] // 2, 2 * D)
    ue2, me2 = _build_gather()(u, m, u2, m2)
    return _mlp(ue2, me2, u.reshape(B, 1), m.reshape(B, 1), W1,
                b1.reshape(1, H), gamma.reshape(1, H), beta.reshape(1, H),
                W2.reshape(1, H), b2.reshape(1, 1))


# speedup vs baseline: 1.6287x; 1.6287x over previous
import functools
import math

import jax
import jax.numpy as jnp
from jax import lax
from jax.experimental import pallas as pl
from jax.experimental.pallas import tpu as pltpu
from jax.experimental.pallas import tpu_sc as plsc

B = 16384
D = 64
H = 1024
BN_EPS = 1e-5
_BN_INV = float(1.0 / math.sqrt(1.0 + BN_EPS))

_NC, _NS = 2, 16
_NW = _NC * _NS
_BPW = B // _NW
_PCH = 256


def _gather_body(u_hbm, m_hbm, uemb_hbm, memb_hbm, ue_out, me_out,
                 uidx_v, midx_v, urows, mrows, sem_u, sem_m):
    wid = lax.axis_index("s") * _NC + lax.axis_index("c")
    base = wid * _BPW
    pltpu.sync_copy(u_hbm.at[pl.ds(base, _BPW)], uidx_v)
    pltpu.sync_copy(m_hbm.at[pl.ds(base, _BPW)], midx_v)

    for p in range(_BPW // _PCH):
        off = p * _PCH

        def _grp(g, carry):
            uv = uidx_v[pl.ds(off + g * 16, 16)]
            mv = midx_v[pl.ds(off + g * 16, 16)]
            for j in range(16):
                r = g * 16 + j
                pltpu.async_copy(uemb_hbm.at[uv[j]], urows.at[r], sem_u)
                pltpu.async_copy(memb_hbm.at[mv[j]], mrows.at[r], sem_m)
            return carry

        lax.fori_loop(0, _PCH // 16, _grp, 0)
        pltpu.make_async_copy(uemb_hbm.at[pl.ds(0, _PCH)], urows, sem_u).wait()
        pltpu.make_async_copy(memb_hbm.at[pl.ds(0, _PCH)], mrows, sem_m).wait()
        dst = pl.ds(base + off, _PCH)
        pltpu.sync_copy(urows, ue_out.at[dst])
        pltpu.sync_copy(mrows, me_out.at[dst])


@functools.cache
def _build_gather():
    mesh = plsc.VectorSubcoreMesh(core_axis_name="c", subcore_axis_name="s",
                                  num_cores=_NC, num_subcores=_NS)
    return pl.kernel(
        _gather_body,
        mesh=mesh,
        out_type=[jax.ShapeDtypeStruct((B, D), jnp.float32),
                  jax.ShapeDtypeStruct((B, D), jnp.float32)],
        scratch_types=[
            pltpu.VMEM((_BPW,), jnp.int32),
            pltpu.VMEM((_BPW,), jnp.int32),
            pltpu.VMEM((_PCH, D), jnp.float32),
            pltpu.VMEM((_PCH, D), jnp.float32),
            pltpu.SemaphoreType.DMA,
            pltpu.SemaphoreType.DMA,
        ],
    )


_BLK = 1024


def _mlp_body(ue_ref, me_ref, w1_ref, b1_ref, gamma_ref,
              beta_ref, w2_ref, b2_ref, out_ref):
    x = jnp.concatenate([ue_ref[...], me_ref[...]], axis=1)
    h = jnp.dot(x, w1_ref[...], preferred_element_type=jnp.float32)
    h = jnp.maximum(h + b1_ref[...], 0.0)
    h = h * (gamma_ref[...] * _BN_INV) + beta_ref[...]
    out = jnp.sum(h * w2_ref[...], axis=1, keepdims=True) + b2_ref[...]
    out_ref[...] = out


_mlp = pl.pallas_call(
    _mlp_body,
    grid=(B // _BLK,),
    in_specs=[
        pl.BlockSpec((_BLK, D), lambda i: (i, 0)),
        pl.BlockSpec((_BLK, D), lambda i: (i, 0)),
        pl.BlockSpec((2 * D, H), lambda i: (0, 0)),
        pl.BlockSpec((1, H), lambda i: (0, 0)),
        pl.BlockSpec((1, H), lambda i: (0, 0)),
        pl.BlockSpec((1, H), lambda i: (0, 0)),
        pl.BlockSpec((1, H), lambda i: (0, 0)),
        pl.BlockSpec((1, 1), lambda i: (0, 0)),
    ],
    out_specs=pl.BlockSpec((_BLK, 1), lambda i: (i, 0)),
    out_shape=jax.ShapeDtypeStruct((B, 1), jnp.float32),
)


def kernel(u, m, u_emb, m_emb, W1, b1, gamma, beta, W2, b2):
    ue, me = _build_gather()(u, m, u_emb, m_emb)
    return _mlp(ue, me, W1,
                b1.reshape(1, H), gamma.reshape(1, H), beta.reshape(1, H),
                W2.reshape(1, H), b2.reshape(1, 1))
